# Initial kernel scaffold; baseline (speedup 1.0000x reference)
import jax
import jax.numpy as jnp
from jax.experimental import pallas as pl


def kernel(indices, embedding_weight):
    raise NotImplementedError("write your pallas kernel here")


# speedup vs baseline: 3.4216x; 3.4216x over previous
import jax
import jax.numpy as jnp
from jax import lax
from jax.experimental import pallas as pl
from jax.experimental.pallas import tpu as pltpu
from jax.experimental.pallas import tpu_sc as plsc

VOCAB = 1000
EMB_DIM = 64
B_TOTAL = 4096 * 200
NC, NS = 2, 16
NW = NC * NS
ROWS = 128
B_PER_W = B_TOTAL // NW
STEPS = B_PER_W // ROWS


def _emb_lookup(indices_3d, table):
    mesh = plsc.VectorSubcoreMesh(core_axis_name="c", subcore_axis_name="s")

    @pl.kernel(
        mesh=mesh,
        out_type=jax.ShapeDtypeStruct((B_TOTAL, EMB_DIM), jnp.float32),
        scratch_types=[
            pltpu.VMEM((STEPS, ROWS), jnp.int32),
            pltpu.VMEM((ROWS, EMB_DIM), jnp.float32),
            pltpu.SemaphoreType.DMA,
        ],
        compiler_params=pltpu.CompilerParams(use_tc_tiling_on_sc=False),
    )
    def k(idx_hbm, table_hbm, out_hbm, idx_v, rows_v, sem):
        wid = lax.axis_index("s") * NC + lax.axis_index("c")
        pltpu.sync_copy(idx_hbm.at[wid], idx_v)
        base = wid * B_PER_W

        def step(j, carry):
            pltpu.async_copy(table_hbm.at[idx_v.at[j]], rows_v, sem).wait()
            pltpu.sync_copy(rows_v, out_hbm.at[pl.ds(base + j * ROWS, ROWS)])
            return carry

        lax.fori_loop(0, STEPS, step, 0)

    return k(indices_3d, table)


def kernel(indices, embedding_weight):
    idx = jnp.asarray(indices, jnp.int32).reshape(NW, STEPS, ROWS)
    out = _emb_lookup(idx, embedding_weight)
    return out.reshape(4096, 200, EMB_DIM)


# speedup vs baseline: 3.5693x; 1.0431x over previous
import jax
import jax.numpy as jnp
from jax import lax
from jax.experimental import pallas as pl
from jax.experimental.pallas import tpu as pltpu
from jax.experimental.pallas import tpu_sc as plsc

VOCAB = 1000
EMB_DIM = 64
B_TOTAL = 4096 * 200
NC, NS = 2, 16
NW = NC * NS
ROWS = 128
K = 4
CHUNK = K * ROWS
B_PER_W = B_TOTAL // NW
NCHUNK = B_PER_W // ROWS
G = B_PER_W // CHUNK


def _emb_lookup(indices_3d, table):
    mesh = plsc.VectorSubcoreMesh(core_axis_name="c", subcore_axis_name="s")

    @pl.kernel(
        mesh=mesh,
        out_type=jax.ShapeDtypeStruct((B_TOTAL, EMB_DIM), jnp.float32),
        scratch_types=[
            pltpu.VMEM((NCHUNK, ROWS), jnp.int32),
            pltpu.VMEM((2, CHUNK, EMB_DIM), jnp.float32),
            pltpu.SemaphoreType.DMA,
            pltpu.SemaphoreType.DMA,
            pltpu.SemaphoreType.DMA,
            pltpu.SemaphoreType.DMA,
        ],
        compiler_params=pltpu.CompilerParams(use_tc_tiling_on_sc=False),
    )
    def k(idx_hbm, table_hbm, out_hbm, idx_v, rows_v, g0, g1, s0, s1):
        wid = lax.axis_index("s") * NC + lax.axis_index("c")
        pltpu.sync_copy(idx_hbm.at[wid], idx_v)
        base = wid * B_PER_W
        gsem = (g0, g1)
        ssem = (s0, s1)

        def fire_gather(g, b):
            for kk in range(K):
                pltpu.async_copy(
                    table_hbm.at[idx_v.at[g * K + kk]],
                    rows_v.at[b].at[pl.ds(kk * ROWS, ROWS)],
                    gsem[b],
                )

        def wait_gather(b):
            pltpu.make_async_copy(
                out_hbm.at[pl.ds(0, CHUNK)], rows_v.at[b], gsem[b]
            ).wait()

        def fire_scatter(g, b):
            pltpu.async_copy(
                rows_v.at[b], out_hbm.at[pl.ds(base + g * CHUNK, CHUNK)],
                ssem[b],
            )

        def wait_scatter(b):
            pltpu.make_async_copy(
                rows_v.at[b], out_hbm.at[pl.ds(0, CHUNK)], ssem[b]
            ).wait()

        fire_gather(0, 0)

        def body(i, carry):
            for b in range(2):
                g = i * 2 + b
                nb = 1 - b

                @pl.when(g >= 1)
                def _():
                    wait_scatter(nb)

                @pl.when(g + 1 < G)
                def _():
                    fire_gather(g + 1, nb)

                wait_gather(b)
                fire_scatter(g, b)
            return carry

        lax.fori_loop(0, G // 2, body, 0)
        wait_scatter(1)

    return k(indices_3d, table)


def kernel(indices, embedding_weight):
    idx = jnp.asarray(indices, jnp.int32).reshape(NW, NCHUNK, ROWS)
    out = _emb_lookup(idx, embedding_weight)
    return out.reshape(4096, 200, EMB_DIM)
